# baseline (speedup 1.0000x reference)
import functools

import jax
import jax.numpy as jnp
from jax import lax
from jax.experimental import pallas as pl
from jax.experimental.pallas import tpu as pltpu
from jax.experimental.pallas import tpu_sc as plsc

NUM_ROWS = 128
ROW_LEN = 100000
LANES = 16
NUM_CORES = 2
NUM_SUBCORES = 16
NUM_WORKERS = NUM_CORES * NUM_SUBCORES
ROWS_PER_WORKER = NUM_ROWS // NUM_WORKERS
UNROLL = 5
CHUNK_OFFS = (0, 51200)
CHUNK_SIZES = (51200, 48800)
CHUNKS_PER_ROW = len(CHUNK_OFFS)
NUM_CHUNKS = ROWS_PER_WORKER * CHUNKS_PER_ROW
BUF_SIZE = max(CHUNK_SIZES)

_NEG_INF = float("-inf")


def _combine(m_a, i_a, m_b, i_b):
    take_b = jnp.logical_or(m_b > m_a,
                            jnp.logical_and(m_b == m_a, i_b < i_a))
    return jnp.where(take_b, m_b, m_a), jnp.where(take_b, i_b, i_a)


def _init_acc():
    iota = lax.broadcasted_iota(jnp.int32, (LANES,), 0)
    ms = tuple(jnp.full((LANES,), _NEG_INF, jnp.float32)
               for _ in range(UNROLL))
    mis = tuple(jnp.zeros((LANES,), jnp.int32) for _ in range(UNROLL))
    idxs = tuple(k * LANES + iota for k in range(UNROLL))
    return ms, mis, idxs


def _consume_chunk(buf_ref, n_iters, carry):
    def body(i, c):
        ms, mis, idxs = c
        ms, mis, idxs = list(ms), list(mis), list(idxs)
        for k in range(UNROLL):
            base = (i * UNROLL + k) * LANES
            v = buf_ref[pl.ds(base, LANES)]
            cmp = v > ms[k]
            ms[k] = jnp.maximum(ms[k], v)
            mis[k] = jnp.where(cmp, idxs[k], mis[k])
            idxs[k] = idxs[k] + (UNROLL * LANES)
        return tuple(ms), tuple(mis), tuple(idxs)

    return lax.fori_loop(0, n_iters, body, carry)


def _finalize_row(carry):
    ms, mis, _ = carry
    m, mi = ms[0], mis[0]
    for k in range(1, UNROLL):
        m, mi = _combine(m, mi, ms[k], mis[k])
    bm = m[0]
    bi = mi[0]
    for l in range(1, LANES):
        v = m[l]
        i = mi[l]
        take = jnp.logical_or(v > bm, jnp.logical_and(v == bm, i < bi))
        bm = jnp.where(take, v, bm)
        bi = jnp.where(take, i, bi)
    return bi


def _build_sc_call():
    mesh = plsc.VectorSubcoreMesh(core_axis_name="c", subcore_axis_name="s",
                                  num_cores=NUM_CORES,
                                  num_subcores=NUM_SUBCORES)

    @functools.partial(
        pl.kernel,
        out_type=jax.ShapeDtypeStruct((NUM_WORKERS, LANES), jnp.int32),
        mesh=mesh,
        scratch_types=[
            pltpu.VMEM((BUF_SIZE,), jnp.float32),
            pltpu.VMEM((BUF_SIZE,), jnp.float32),
            pltpu.VMEM((LANES,), jnp.int32),
            pltpu.VMEM((LANES,), jnp.int32),
            pltpu.SemaphoreType.DMA,
            pltpu.SemaphoreType.DMA,
            pltpu.SemaphoreType.DMA,
        ],
        compiler_params=pltpu.CompilerParams(use_tc_tiling_on_sc=False),
    )
    def sc_kernel(y_hbm, table_hbm, out_hbm, buf0, buf1, idx_buf, word_buf,
                  sem0, sem1, gsem):
        wid = lax.axis_index("s") * NUM_CORES + lax.axis_index("c")
        base_row = wid * ROWS_PER_WORKER
        iota = lax.broadcasted_iota(jnp.int32, (LANES,), 0)
        bufs = (buf0, buf1)
        sems = (sem0, sem1)

        def start_copy(c):
            r, h = divmod(c, CHUNKS_PER_ROW)
            return pltpu.async_copy(
                y_hbm.at[base_row + r].at[pl.ds(CHUNK_OFFS[h],
                                                CHUNK_SIZES[h])],
                bufs[c % 2].at[pl.ds(0, CHUNK_SIZES[h])], sems[c % 2])

        res_vec = jnp.zeros((LANES,), jnp.int32)
        carry = _init_acc()
        handle = start_copy(0)
        for c in range(NUM_CHUNKS):
            next_handle = start_copy(c + 1) if c + 1 < NUM_CHUNKS else None
            handle.wait()
            h = c % CHUNKS_PER_ROW
            carry = _consume_chunk(bufs[c % 2],
                                   CHUNK_SIZES[h] // (UNROLL * LANES), carry)
            r, h = divmod(c, CHUNKS_PER_ROW)
            if h == CHUNKS_PER_ROW - 1:
                a = _finalize_row(carry)
                res_vec = jnp.where(iota == r, a, res_vec)
                if c + 1 < NUM_CHUNKS:
                    carry = _init_acc()
            handle = next_handle

        idx_buf[...] = res_vec
        pltpu.async_copy(table_hbm.at[idx_buf], word_buf, gsem).wait()
        pltpu.sync_copy(word_buf, out_hbm.at[wid])

    return sc_kernel


_sc_call = _build_sc_call()


@jax.jit
def kernel(y_pred, word_table):
    staged = _sc_call(y_pred, word_table)
    return staged[:, :ROWS_PER_WORKER].reshape(-1)


# speedup vs baseline: 1.7006x; 1.7006x over previous
import functools

import jax
import jax.numpy as jnp
from jax import lax
from jax.experimental import pallas as pl
from jax.experimental.pallas import tpu as pltpu
from jax.experimental.pallas import tpu_sc as plsc

NUM_ROWS = 128
ROW_LEN = 100000
LANES = 16
NUM_CORES = 2
NUM_SUBCORES = 16
NUM_WORKERS = NUM_CORES * NUM_SUBCORES
ROWS_PER_WORKER = NUM_ROWS // NUM_WORKERS
UNROLL = 5
STEP = UNROLL * LANES
CHUNK = 49920
ITERS_PER_CHUNK = CHUNK // STEP
MAIN_LEN = 2 * CHUNK
TAIL_LEN = ROW_LEN - MAIN_LEN
TAIL_STEPS = TAIL_LEN // STEP
NUM_BIG = ROWS_PER_WORKER * 2

_NEG_INF = float("-inf")


def _acc_init():
    ms = tuple(jnp.full((LANES,), _NEG_INF, jnp.float32)
               for _ in range(UNROLL))
    its = tuple(jnp.zeros((LANES,), jnp.int32) for _ in range(UNROLL))
    return ms, its


def _acc_step(buf_ref, base, it_vec, carry):
    ms, its = carry
    ms, its = list(ms), list(its)
    for k in range(UNROLL):
        v = buf_ref[pl.ds(base + k * LANES, LANES)]
        cmp = v > ms[k]
        ms[k] = jnp.maximum(ms[k], v)
        its[k] = jnp.where(cmp, it_vec, its[k])
    return tuple(ms), tuple(its)


def _consume_chunk(buf_ref, it_base, carry):
    def body(i, c):
        it_vec = jnp.broadcast_to(it_base + i, (LANES,)).astype(jnp.int32)
        return _acc_step(buf_ref, i * STEP, it_vec, c)

    return lax.fori_loop(0, ITERS_PER_CHUNK, body, carry)


def _finalize_row(carry):
    ms, its = carry
    iota = lax.broadcasted_iota(jnp.int32, (LANES,), 0)
    m, mi = ms[0], its[0] * STEP + iota
    for k in range(1, UNROLL):
        mk = ms[k]
        mik = its[k] * STEP + (k * LANES) + iota
        take = jnp.logical_or(mk > m, jnp.logical_and(mk == m, mik < mi))
        m = jnp.where(take, mk, m)
        mi = jnp.where(take, mik, mi)
    bm = m[0]
    bi = mi[0]
    for l in range(1, LANES):
        v = m[l]
        i = mi[l]
        take = jnp.logical_or(v > bm, jnp.logical_and(v == bm, i < bi))
        bm = jnp.where(take, v, bm)
        bi = jnp.where(take, i, bi)
    return bi


def _build_sc_call():
    mesh = plsc.VectorSubcoreMesh(core_axis_name="c", subcore_axis_name="s",
                                  num_cores=NUM_CORES,
                                  num_subcores=NUM_SUBCORES)

    @functools.partial(
        pl.kernel,
        out_type=jax.ShapeDtypeStruct((NUM_WORKERS, LANES), jnp.int32),
        mesh=mesh,
        scratch_types=[
            pltpu.VMEM((CHUNK,), jnp.float32),
            pltpu.VMEM((CHUNK,), jnp.float32),
            pltpu.VMEM((TAIL_LEN,), jnp.float32),
            pltpu.VMEM((LANES,), jnp.int32),
            pltpu.VMEM((LANES,), jnp.int32),
            pltpu.SemaphoreType.DMA,
            pltpu.SemaphoreType.DMA,
            pltpu.SemaphoreType.DMA,
            pltpu.SemaphoreType.DMA,
        ],
    )
    def sc_kernel(y_hbm, tail_hbm, table_hbm, out_hbm, buf0, buf1, tbuf,
                  idx_buf, word_buf, sem0, sem1, tsem, gsem):
        wid = lax.axis_index("s") * NUM_CORES + lax.axis_index("c")
        base_row = wid * ROWS_PER_WORKER
        iota = lax.broadcasted_iota(jnp.int32, (LANES,), 0)
        bufs = (buf0, buf1)
        sems = (sem0, sem1)

        def start_big(c):
            r, h = divmod(c, 2)
            return pltpu.async_copy(
                y_hbm.at[base_row + r].at[pl.ds(h * CHUNK, CHUNK)],
                bufs[c % 2], sems[c % 2])

        def start_tail(r):
            return pltpu.async_copy(tail_hbm.at[base_row + r], tbuf, tsem)

        res_vec = jnp.zeros((LANES,), jnp.int32)
        carry = _acc_init()
        big_handles = {0: start_big(0), 1: start_big(1)}
        tail_handle = start_tail(0)
        for c in range(NUM_BIG):
            big_handles.pop(c).wait()
            carry = _consume_chunk(bufs[c % 2], (c % 2) * ITERS_PER_CHUNK,
                                   carry)
            if c + 2 < NUM_BIG:
                big_handles[c + 2] = start_big(c + 2)
            if c % 2 == 1:
                r = c // 2
                tail_handle.wait()
                for t in range(TAIL_STEPS):
                    it = 2 * ITERS_PER_CHUNK + t
                    it_vec = jnp.broadcast_to(jnp.int32(it), (LANES,))
                    carry = _acc_step(tbuf, t * STEP, it_vec, carry)
                a = _finalize_row(carry)
                res_vec = jnp.where(iota == r, a, res_vec)
                if r + 1 < ROWS_PER_WORKER:
                    tail_handle = start_tail(r + 1)
                    carry = _acc_init()

        idx_buf[...] = res_vec
        pltpu.async_copy(table_hbm.at[idx_buf], word_buf, gsem).wait()
        pltpu.sync_copy(word_buf, out_hbm.at[wid])

    return sc_kernel


_sc_call = _build_sc_call()


@jax.jit
def kernel(y_pred, word_table):
    y_tail = lax.slice(y_pred, (0, MAIN_LEN), (NUM_ROWS, ROW_LEN))
    staged = _sc_call(y_pred, y_tail, word_table)
    return staged[:, :ROWS_PER_WORKER].reshape(-1)


# speedup vs baseline: 2.5608x; 1.5058x over previous
import functools

import jax
import jax.numpy as jnp
from jax import lax
from jax.experimental import pallas as pl
from jax.experimental.pallas import tpu as pltpu
from jax.experimental.pallas import tpu_sc as plsc

NUM_ROWS = 128
ROW_LEN = 100000
LANES = 16
NUM_CORES = 2
NUM_SUBCORES = 16
NUM_WORKERS = NUM_CORES * NUM_SUBCORES
GROUPS = NUM_ROWS // LANES
STRIPE = 3128
CHUNK_V = 184
CHUNKS = STRIPE // CHUNK_V
LAST_STRIPE_BASE = ROW_LEN - STRIPE

_NEG_INF = float("-inf")


def _consume_chunk(buf_ref, idx_base, carry):
    def body(j, c):
        ms, its = c
        ms, its = list(ms), list(its)
        it_vec = jnp.broadcast_to(idx_base + j, (LANES,)).astype(jnp.int32)
        for g in range(GROUPS):
            v = buf_ref[j, pl.ds(g * LANES, LANES)]
            cmp = v > ms[g]
            ms[g] = jnp.maximum(ms[g], v)
            its[g] = jnp.where(cmp, it_vec, its[g])
        return tuple(ms), tuple(its)

    return lax.fori_loop(0, CHUNK_V, body, carry)


def _build_sc_call():
    mesh = plsc.VectorSubcoreMesh(core_axis_name="c", subcore_axis_name="s",
                                  num_cores=NUM_CORES,
                                  num_subcores=NUM_SUBCORES)

    @functools.partial(
        pl.kernel,
        out_type=(
            jax.ShapeDtypeStruct((NUM_WORKERS, GROUPS, LANES), jnp.float32),
            jax.ShapeDtypeStruct((NUM_WORKERS, GROUPS, LANES), jnp.int32),
        ),
        mesh=mesh,
        scratch_types=[
            pltpu.VMEM((CHUNK_V, NUM_ROWS), jnp.float32),
            pltpu.VMEM((CHUNK_V, NUM_ROWS), jnp.float32),
            pltpu.VMEM((GROUPS, LANES), jnp.float32),
            pltpu.VMEM((GROUPS, LANES), jnp.int32),
            pltpu.VMEM((NUM_SUBCORES, LANES), jnp.float32),
            pltpu.VMEM((NUM_SUBCORES, LANES), jnp.int32),
            pltpu.VMEM((LANES,), jnp.int32),
            pltpu.VMEM((LANES,), jnp.int32),
            pltpu.VMEM((LANES,), jnp.float32),
            pltpu.VMEM_SHARED((GROUPS, NUM_SUBCORES, LANES), jnp.float32),
            pltpu.VMEM_SHARED((GROUPS, NUM_SUBCORES, LANES), jnp.int32),
            pltpu.SemaphoreType.DMA,
            pltpu.SemaphoreType.DMA,
            pltpu.SemaphoreType.DMA,
        ],
    )
    def sc_kernel(yt_hbm, table_hbm, out_val_hbm, out_word_hbm,
                  buf0, buf1, cand_val, cand_idx, merge_val, merge_idx,
                  idx_buf, word_buf, val_buf, stage_val, stage_idx,
                  sem0, sem1, gsem):
        core = lax.axis_index("c")
        sub = lax.axis_index("s")
        stripe_rank = core * NUM_SUBCORES + sub
        sb = jnp.minimum(stripe_rank * STRIPE, LAST_STRIPE_BASE)
        sb = pl.multiple_of(sb, 8)
        bufs = (buf0, buf1)
        sems = (sem0, sem1)

        def start_copy(c):
            return pltpu.async_copy(
                yt_hbm.at[pl.ds(sb + c * CHUNK_V, CHUNK_V)],
                bufs[c % 2], sems[c % 2])

        ms = tuple(jnp.full((LANES,), _NEG_INF, jnp.float32)
                   for _ in range(GROUPS))
        its = tuple(jnp.zeros((LANES,), jnp.int32) for _ in range(GROUPS))
        carry = (ms, its)
        handle = start_copy(0)
        for c in range(CHUNKS):
            next_handle = start_copy(c + 1) if c + 1 < CHUNKS else None
            handle.wait()
            carry = _consume_chunk(bufs[c % 2], sb + c * CHUNK_V, carry)
            handle = next_handle
        ms, its = carry

        for g in range(GROUPS):
            cand_val[g, ...] = ms[g]
            cand_idx[g, ...] = its[g]
        pltpu.sync_copy(cand_val, out_val_hbm.at[stripe_rank])
        pltpu.sync_copy(cand_idx, out_word_hbm.at[stripe_rank])

    return sc_kernel


_sc_call = _build_sc_call()


@jax.jit
def kernel(y_pred, word_table):
    vals, idxs = _sc_call(y_pred.T, word_table)
    vals = vals.reshape(NUM_WORKERS, NUM_ROWS)
    idxs = idxs.reshape(NUM_WORKERS, NUM_ROWS)
    order = jnp.argsort(idxs, axis=0)
    vals = jnp.take_along_axis(vals, order, axis=0)
    idxs = jnp.take_along_axis(idxs, order, axis=0)
    best = jnp.argmax(vals, axis=0)
    win = jnp.take_along_axis(idxs, best[None], axis=0)[0]
    return jnp.take(word_table, win)


# speedup vs baseline: 3.1572x; 1.2329x over previous
import functools

import jax
import jax.numpy as jnp
from jax import lax
from jax.experimental import pallas as pl
from jax.experimental.pallas import tpu as pltpu
from jax.experimental.pallas import tpu_sc as plsc

NUM_ROWS = 128
ROW_LEN = 100000
LANES = 16
NUM_CORES = 2
NUM_SUBCORES = 16
NUM_WORKERS = NUM_CORES * NUM_SUBCORES
GROUPS = NUM_ROWS // LANES
STRIPE = 3128
CHUNK_V = 184
CHUNKS = STRIPE // CHUNK_V
LAST_STRIPE_BASE = ROW_LEN - STRIPE

_NEG_INF = float("-inf")


def _consume_chunk(buf_ref, idx_base, carry):
    def body(j, c):
        ms, its = c
        ms, its = list(ms), list(its)
        it_vec = jnp.broadcast_to(idx_base + j, (LANES,)).astype(jnp.int32)
        for g in range(GROUPS):
            v = buf_ref[j, pl.ds(g * LANES, LANES)]
            cmp = v > ms[g]
            ms[g] = jnp.maximum(ms[g], v)
            its[g] = jnp.where(cmp, it_vec, its[g])
        return tuple(ms), tuple(its)

    return lax.fori_loop(0, CHUNK_V, body, carry)


def _build_sc_call():
    mesh = plsc.VectorSubcoreMesh(core_axis_name="c", subcore_axis_name="s",
                                  num_cores=NUM_CORES,
                                  num_subcores=NUM_SUBCORES)

    @functools.partial(
        pl.kernel,
        out_type=(
            jax.ShapeDtypeStruct((NUM_CORES * GROUPS, LANES), jnp.float32),
            jax.ShapeDtypeStruct((NUM_CORES * GROUPS, LANES), jnp.int32),
            jax.ShapeDtypeStruct((NUM_WORKERS, GROUPS, LANES), jnp.float32),
            jax.ShapeDtypeStruct((NUM_WORKERS, GROUPS, LANES), jnp.int32),
        ),
        mesh=mesh,
        scratch_types=[
            pltpu.VMEM((CHUNK_V, NUM_ROWS), jnp.float32),
            pltpu.VMEM((CHUNK_V, NUM_ROWS), jnp.float32),
            pltpu.VMEM((GROUPS, LANES), jnp.float32),
            pltpu.VMEM((GROUPS, LANES), jnp.int32),
            pltpu.VMEM((NUM_SUBCORES, GROUPS, LANES), jnp.float32),
            pltpu.VMEM((NUM_SUBCORES, GROUPS, LANES), jnp.int32),
            pltpu.VMEM((LANES,), jnp.int32),
            pltpu.VMEM((LANES,), jnp.int32),
            pltpu.VMEM((LANES,), jnp.float32),
            pltpu.SemaphoreType.DMA,
            pltpu.SemaphoreType.DMA,
            pltpu.SemaphoreType.DMA,
        ],
    )
    def sc_kernel(yt_hbm, table_hbm, out_val_hbm, out_word_hbm,
                  stage_val_hbm, stage_idx_hbm,
                  buf0, buf1, cand_val, cand_idx, merge_val, merge_idx,
                  idx_buf, word_buf, val_buf,
                  sem0, sem1, gsem):
        core = lax.axis_index("c")
        sub = lax.axis_index("s")
        stripe_rank = core * NUM_SUBCORES + sub
        sb = jnp.minimum(stripe_rank * STRIPE, LAST_STRIPE_BASE)
        sb = pl.multiple_of(sb, 8)
        bufs = (buf0, buf1)
        sems = (sem0, sem1)

        def start_copy(c):
            return pltpu.async_copy(
                yt_hbm.at[pl.ds(sb + c * CHUNK_V, CHUNK_V)],
                bufs[c % 2], sems[c % 2])

        ms = tuple(jnp.full((LANES,), _NEG_INF, jnp.float32)
                   for _ in range(GROUPS))
        its = tuple(jnp.zeros((LANES,), jnp.int32) for _ in range(GROUPS))
        carry = (ms, its)
        handle = start_copy(0)
        for c in range(CHUNKS):
            next_handle = start_copy(c + 1) if c + 1 < CHUNKS else None
            handle.wait()
            carry = _consume_chunk(bufs[c % 2], sb + c * CHUNK_V, carry)
            handle = next_handle
        ms, its = carry

        for g in range(GROUPS):
            cand_val[g, ...] = ms[g]
            cand_idx[g, ...] = its[g]
        pltpu.sync_copy(cand_val, stage_val_hbm.at[stripe_rank])
        pltpu.sync_copy(cand_idx, stage_idx_hbm.at[stripe_rank])
        plsc.subcore_barrier()

        @pl.when(sub < GROUPS)
        def _merge():
            g = sub
            pltpu.sync_copy(
                stage_val_hbm.at[pl.ds(core * NUM_SUBCORES, NUM_SUBCORES)],
                merge_val)
            pltpu.sync_copy(
                stage_idx_hbm.at[pl.ds(core * NUM_SUBCORES, NUM_SUBCORES)],
                merge_idx)
            bv = merge_val[0, g, ...]
            bi = merge_idx[0, g, ...]
            for t in range(1, NUM_SUBCORES):
                v = merge_val[t, g, ...]
                i = merge_idx[t, g, ...]
                take = jnp.logical_or(
                    v > bv, jnp.logical_and(v == bv, i < bi))
                bv = jnp.where(take, v, bv)
                bi = jnp.where(take, i, bi)
            idx_buf[...] = bi
            val_buf[...] = bv
            pltpu.async_copy(table_hbm.at[idx_buf], word_buf, gsem).wait()
            out_row = core * GROUPS + g
            pltpu.sync_copy(val_buf, out_val_hbm.at[out_row])
            pltpu.sync_copy(word_buf, out_word_hbm.at[out_row])

    return sc_kernel


_sc_call = _build_sc_call()


@jax.jit
def kernel(y_pred, word_table):
    vals, words, _, _ = _sc_call(y_pred.T, word_table)
    v0 = vals[:GROUPS].reshape(-1)
    v1 = vals[GROUPS:].reshape(-1)
    w0 = words[:GROUPS].reshape(-1)
    w1 = words[GROUPS:].reshape(-1)
    return jnp.where(v0 >= v1, w0, w1)


# speedup vs baseline: 3.5559x; 1.1263x over previous
import functools

import jax
import jax.numpy as jnp
from jax import lax
from jax.experimental import pallas as pl
from jax.experimental.pallas import tpu as pltpu
from jax.experimental.pallas import tpu_sc as plsc

NUM_ROWS = 128
ROW_LEN = 100000
LANES = 16
NUM_CORES = 2
NUM_SUBCORES = 16
NUM_WORKERS = NUM_CORES * NUM_SUBCORES
GROUPS = NUM_ROWS // LANES
STRIPE = 3128
CHUNK_V = 184
CHUNKS = STRIPE // CHUNK_V
LAST_STRIPE_BASE = ROW_LEN - STRIPE

_NEG_INF = float("-inf")


def _consume_chunk(buf_ref, idx_base, carry):
    def body(j, c):
        ms, its = c
        ms, its = list(ms), list(its)
        it_vec = jnp.broadcast_to(idx_base + j, (LANES,)).astype(jnp.int32)
        for g in range(GROUPS):
            v = buf_ref[j, pl.ds(g * LANES, LANES)]
            cmp = v > ms[g]
            ms[g] = jnp.maximum(ms[g], v)
            its[g] = jnp.where(cmp, it_vec, its[g])
        return tuple(ms), tuple(its)

    return lax.fori_loop(0, CHUNK_V, body, carry)


def _build_sc_call():
    mesh = plsc.VectorSubcoreMesh(core_axis_name="c", subcore_axis_name="s",
                                  num_cores=NUM_CORES,
                                  num_subcores=NUM_SUBCORES)

    @functools.partial(
        pl.kernel,
        out_type=(
            jax.ShapeDtypeStruct((NUM_CORES * GROUPS, LANES), jnp.float32),
            jax.ShapeDtypeStruct((NUM_CORES * GROUPS, LANES), jnp.int32),
            jax.ShapeDtypeStruct((NUM_WORKERS, GROUPS, LANES), jnp.float32),
            jax.ShapeDtypeStruct((NUM_WORKERS, GROUPS, LANES), jnp.int32),
        ),
        mesh=mesh,
        scratch_types=[
            pltpu.VMEM((CHUNK_V, NUM_ROWS), jnp.float32),
            pltpu.VMEM((CHUNK_V, NUM_ROWS), jnp.float32),
            pltpu.VMEM((CHUNK_V, NUM_ROWS), jnp.float32),
            pltpu.VMEM((CHUNK_V, NUM_ROWS), jnp.float32),
            pltpu.VMEM((GROUPS, LANES), jnp.float32),
            pltpu.VMEM((GROUPS, LANES), jnp.int32),
            pltpu.VMEM((NUM_SUBCORES, GROUPS, LANES), jnp.float32),
            pltpu.VMEM((NUM_SUBCORES, GROUPS, LANES), jnp.int32),
            pltpu.VMEM((LANES,), jnp.int32),
            pltpu.VMEM((LANES,), jnp.int32),
            pltpu.VMEM((LANES,), jnp.float32),
            pltpu.SemaphoreType.DMA,
            pltpu.SemaphoreType.DMA,
            pltpu.SemaphoreType.DMA,
            pltpu.SemaphoreType.DMA,
            pltpu.SemaphoreType.DMA,
        ],
    )
    def sc_kernel(yt_hbm, table_hbm, out_val_hbm, out_word_hbm,
                  stage_val_hbm, stage_idx_hbm,
                  buf0, buf1, buf2, buf3, cand_val, cand_idx,
                  merge_val, merge_idx,
                  idx_buf, word_buf, val_buf,
                  sem0, sem1, sem2, sem3, gsem):
        core = lax.axis_index("c")
        sub = lax.axis_index("s")
        stripe_rank = core * NUM_SUBCORES + sub
        sb = jnp.minimum(stripe_rank * STRIPE, LAST_STRIPE_BASE)
        sb = pl.multiple_of(sb, 8)
        bufs = (buf0, buf1, buf2, buf3)
        sems = (sem0, sem1, sem2, sem3)
        NBUF = len(bufs)

        def start_copy(c):
            return pltpu.async_copy(
                yt_hbm.at[pl.ds(sb + c * CHUNK_V, CHUNK_V)],
                bufs[c % NBUF], sems[c % NBUF])

        ms = tuple(jnp.full((LANES,), _NEG_INF, jnp.float32)
                   for _ in range(GROUPS))
        its = tuple(jnp.zeros((LANES,), jnp.int32) for _ in range(GROUPS))
        carry = (ms, its)
        handles = {c: start_copy(c) for c in range(NBUF - 1)}
        for c in range(CHUNKS):
            handles.pop(c).wait()
            if c + NBUF - 1 < CHUNKS:
                handles[c + NBUF - 1] = start_copy(c + NBUF - 1)
            carry = _consume_chunk(bufs[c % NBUF], sb + c * CHUNK_V, carry)
        ms, its = carry

        for g in range(GROUPS):
            cand_val[g, ...] = ms[g]
            cand_idx[g, ...] = its[g]
        pltpu.sync_copy(cand_val, stage_val_hbm.at[stripe_rank])
        pltpu.sync_copy(cand_idx, stage_idx_hbm.at[stripe_rank])
        plsc.subcore_barrier()

        @pl.when(sub < GROUPS)
        def _merge():
            g = sub
            pltpu.sync_copy(
                stage_val_hbm.at[pl.ds(core * NUM_SUBCORES, NUM_SUBCORES)],
                merge_val)
            pltpu.sync_copy(
                stage_idx_hbm.at[pl.ds(core * NUM_SUBCORES, NUM_SUBCORES)],
                merge_idx)
            bv = merge_val[0, g, ...]
            bi = merge_idx[0, g, ...]
            for t in range(1, NUM_SUBCORES):
                v = merge_val[t, g, ...]
                i = merge_idx[t, g, ...]
                take = jnp.logical_or(
                    v > bv, jnp.logical_and(v == bv, i < bi))
                bv = jnp.where(take, v, bv)
                bi = jnp.where(take, i, bi)
            idx_buf[...] = bi
            val_buf[...] = bv
            pltpu.async_copy(table_hbm.at[idx_buf], word_buf, gsem).wait()
            out_row = core * GROUPS + g
            pltpu.sync_copy(val_buf, out_val_hbm.at[out_row])
            pltpu.sync_copy(word_buf, out_word_hbm.at[out_row])

    return sc_kernel


_sc_call = _build_sc_call()


@jax.jit
def kernel(y_pred, word_table):
    vals, words, _, _ = _sc_call(y_pred.T, word_table)
    v0 = vals[:GROUPS].reshape(-1)
    v1 = vals[GROUPS:].reshape(-1)
    w0 = words[:GROUPS].reshape(-1)
    w1 = words[GROUPS:].reshape(-1)
    return jnp.where(v0 >= v1, w0, w1)
